# initial kernel scaffold (unmeasured)
import jax
import jax.numpy as jnp
from jax import lax
from jax.experimental import pallas as pl
from jax.experimental.pallas import tpu as pltpu


def kernel(
    x,
):
    def body(*refs):
        pass

    out_shape = jax.ShapeDtypeStruct(..., jnp.float32)
    return pl.pallas_call(body, out_shape=out_shape)(...)


# baseline (device time: 16697 ns/iter reference)
import jax
import jax.numpy as jnp
from jax import lax
from jax.experimental import pallas as pl
from jax.experimental.pallas import tpu as pltpu

N_DEV = 16
N_ROUNDS = 4


def kernel(x):
    m, n = x.shape

    def body(x_ref, out_ref, v_ref, e_ref, send_buf, recv_buf,
             send_sems, recv_sems):
        my = lax.axis_index("i")

        acc = x_ref[:, :]
        shift = 1
        while shift < m:
            shifted = jnp.concatenate(
                [jnp.ones((shift, n), jnp.float32), acc[: m - shift, :]],
                axis=0,
            )
            acc = acc * shifted
            shift *= 2

        v_ref[:, :] = acc[m - 1 : m, :]
        e_ref[:, :] = jnp.ones((1, n), jnp.float32)

        for k in range(N_ROUNDS):
            d = 1 << k
            sends = my + d < N_DEV
            recvs = my >= d

            @pl.when(sends)
            def _():
                send_buf[:, :] = v_ref[:, :]
                rdma = pltpu.make_async_remote_copy(
                    src_ref=send_buf,
                    dst_ref=recv_buf.at[k],
                    send_sem=send_sems.at[k],
                    recv_sem=recv_sems.at[k],
                    device_id=(my + d,),
                    device_id_type=pl.DeviceIdType.MESH,
                )
                rdma.start()

            @pl.when(recvs)
            def _():
                recv = pltpu.make_async_remote_copy(
                    src_ref=send_buf,
                    dst_ref=recv_buf.at[k],
                    send_sem=send_sems.at[k],
                    recv_sem=recv_sems.at[k],
                    device_id=(my - d,),
                    device_id_type=pl.DeviceIdType.MESH,
                )
                recv.wait_recv()
                r = recv_buf[k, :, :]
                e_ref[:, :] = e_ref[:, :] * r
                v_ref[:, :] = v_ref[:, :] * r

            @pl.when(sends)
            def _():
                w = pltpu.make_async_remote_copy(
                    src_ref=send_buf,
                    dst_ref=recv_buf.at[k],
                    send_sem=send_sems.at[k],
                    recv_sem=recv_sems.at[k],
                    device_id=(my + d,),
                    device_id_type=pl.DeviceIdType.MESH,
                )
                w.wait_send()

        out_ref[:, :] = acc * e_ref[:, :]

    return pl.pallas_call(
        body,
        out_shape=jax.ShapeDtypeStruct((m, n), jnp.float32),
        in_specs=[pl.BlockSpec(memory_space=pltpu.VMEM)],
        out_specs=pl.BlockSpec(memory_space=pltpu.VMEM),
        scratch_shapes=[
            pltpu.VMEM((1, n), jnp.float32),
            pltpu.VMEM((1, n), jnp.float32),
            pltpu.VMEM((1, n), jnp.float32),
            pltpu.VMEM((N_ROUNDS, 1, n), jnp.float32),
            pltpu.SemaphoreType.DMA((N_ROUNDS,)),
            pltpu.SemaphoreType.DMA((N_ROUNDS,)),
        ],
    )(x)


# device time: 14942 ns/iter; 1.1175x vs baseline; 1.1175x over previous
import jax
import jax.numpy as jnp
from jax import lax
from jax.experimental import pallas as pl
from jax.experimental.pallas import tpu as pltpu

N_DEV = 16
N_ROUNDS = 4


def kernel(x):
    m, n = x.shape

    def body(x_ref, out_ref, v_ref, e_ref, send_buf, recv_buf,
             send_sems, recv_sems):
        my = lax.axis_index("i")

        t = x_ref[:, :]
        h = m
        while h > 1:
            h //= 2
            t = t[:h, :] * t[h:, :]
        v_ref[:, :] = t
        e_ref[:, :] = jnp.ones((1, n), jnp.float32)

        def start_round(k):
            d = 1 << k

            @pl.when(my + d < N_DEV)
            def _():
                send_buf[:, :] = v_ref[:, :]
                rdma = pltpu.make_async_remote_copy(
                    src_ref=send_buf,
                    dst_ref=recv_buf.at[k],
                    send_sem=send_sems.at[k],
                    recv_sem=recv_sems.at[k],
                    device_id=(my + d,),
                    device_id_type=pl.DeviceIdType.MESH,
                )
                rdma.start()

        def finish_round(k):
            d = 1 << k

            @pl.when(my >= d)
            def _():
                recv = pltpu.make_async_remote_copy(
                    src_ref=send_buf,
                    dst_ref=recv_buf.at[k],
                    send_sem=send_sems.at[k],
                    recv_sem=recv_sems.at[k],
                    device_id=(my - d,),
                    device_id_type=pl.DeviceIdType.MESH,
                )
                recv.wait_recv()
                r = recv_buf[k, :, :]
                e_ref[:, :] = e_ref[:, :] * r
                v_ref[:, :] = v_ref[:, :] * r

            @pl.when(my + d < N_DEV)
            def _():
                w = pltpu.make_async_remote_copy(
                    src_ref=send_buf,
                    dst_ref=recv_buf.at[k],
                    send_sem=send_sems.at[k],
                    recv_sem=recv_sems.at[k],
                    device_id=(my + d,),
                    device_id_type=pl.DeviceIdType.MESH,
                )
                w.wait_send()

        def cumprod_steps(acc, shifts):
            for shift in shifts:
                shifted = jnp.concatenate(
                    [jnp.ones((shift, n), jnp.float32), acc[: m - shift, :]],
                    axis=0,
                )
                acc = acc * shifted
            return acc

        acc = x_ref[:, :]
        start_round(0)
        acc = cumprod_steps(acc, [1, 2, 4])
        finish_round(0)
        start_round(1)
        acc = cumprod_steps(acc, [8, 16, 32])
        finish_round(1)
        start_round(2)
        acc = cumprod_steps(acc, [64, 128])
        finish_round(2)
        start_round(3)
        acc = cumprod_steps(acc, [256, 512])
        finish_round(3)

        out_ref[:, :] = acc * e_ref[:, :]

    return pl.pallas_call(
        body,
        out_shape=jax.ShapeDtypeStruct((m, n), jnp.float32),
        in_specs=[pl.BlockSpec(memory_space=pltpu.VMEM)],
        out_specs=pl.BlockSpec(memory_space=pltpu.VMEM),
        scratch_shapes=[
            pltpu.VMEM((1, n), jnp.float32),
            pltpu.VMEM((1, n), jnp.float32),
            pltpu.VMEM((1, n), jnp.float32),
            pltpu.VMEM((N_ROUNDS, 1, n), jnp.float32),
            pltpu.SemaphoreType.DMA((N_ROUNDS,)),
            pltpu.SemaphoreType.DMA((N_ROUNDS,)),
        ],
    )(x)


# device time: 8442 ns/iter; 1.9778x vs baseline; 1.7700x over previous
import jax
import jax.numpy as jnp
from jax import lax
from jax.experimental import pallas as pl
from jax.experimental.pallas import tpu as pltpu

N_DEV = 16
OFFS_A = (1, 2, 3)
OFFS_B = (4, 8, 12)


def kernel(x):
    m, n = x.shape

    def body(x_ref, out_ref, e_ref, sa_buf, sb_buf, ra_buf, rb_buf,
             sa_sems, ra_sems, sb_sems, rb_sems, acka_sems, ackb_sems):
        my = lax.axis_index("i")

        for j, o in enumerate(OFFS_A):
            @pl.when(my >= o)
            def _(j=j, o=o):
                pl.semaphore_signal(
                    acka_sems.at[j], inc=1,
                    device_id=(my - o,),
                    device_id_type=pl.DeviceIdType.MESH,
                )
        for j, o in enumerate(OFFS_B):
            @pl.when(my >= o)
            def _(j=j, o=o):
                pl.semaphore_signal(
                    ackb_sems.at[j], inc=1,
                    device_id=(my - o,),
                    device_id_type=pl.DeviceIdType.MESH,
                )

        barrier_sem = pltpu.get_barrier_semaphore()
        pl.semaphore_signal(barrier_sem, inc=1)
        pl.semaphore_wait(barrier_sem, 1)

        t = x_ref[:, :]
        h = m
        while h > 1:
            h //= 2
            t = t[:h, :] * t[h:, :]
        sa_buf[:, :] = t

        def copy(src, dst, j, send_sems, recv_sems, target):
            return pltpu.make_async_remote_copy(
                src_ref=src,
                dst_ref=dst.at[j],
                send_sem=send_sems.at[j],
                recv_sem=recv_sems.at[j],
                device_id=(target,),
                device_id_type=pl.DeviceIdType.MESH,
            )

        def cumprod_steps(acc, shifts):
            for shift in shifts:
                shifted = jnp.concatenate(
                    [jnp.ones((shift, n), jnp.float32), acc[: m - shift, :]],
                    axis=0,
                )
                acc = acc * shifted
            return acc

        acc = x_ref[:, :]
        acc = cumprod_steps(acc, (1, 2))

        for j, o in enumerate(OFFS_A):
            @pl.when(my + o < N_DEV)
            def _(j=j, o=o):
                pl.semaphore_wait(acka_sems.at[j], 1)
                copy(sa_buf, ra_buf, j, sa_sems, ra_sems, my + o).start()

        acc = cumprod_steps(acc, (4, 8))

        e_ref[:, :] = jnp.ones((1, n), jnp.float32)
        for j, o in enumerate(OFFS_A):
            @pl.when(my >= o)
            def _(j=j, o=o):
                copy(sa_buf, ra_buf, j, sa_sems, ra_sems, my - o).wait_recv()
                e_ref[:, :] = e_ref[:, :] * ra_buf[j, :, :]

        sb_buf[:, :] = sa_buf[:, :] * e_ref[:, :]

        for j, o in enumerate(OFFS_B):
            @pl.when(my + o < N_DEV)
            def _(j=j, o=o):
                pl.semaphore_wait(ackb_sems.at[j], 1)
                copy(sb_buf, rb_buf, j, sb_sems, rb_sems, my + o).start()

        acc = cumprod_steps(acc, (16, 32, 64, 128, 256, 512))

        for j, o in enumerate(OFFS_B):
            @pl.when(my >= o)
            def _(j=j, o=o):
                copy(sb_buf, rb_buf, j, sb_sems, rb_sems, my - o).wait_recv()
                e_ref[:, :] = e_ref[:, :] * rb_buf[j, :, :]

        for j, o in enumerate(OFFS_A):
            @pl.when(my + o < N_DEV)
            def _(j=j, o=o):
                copy(sa_buf, ra_buf, j, sa_sems, ra_sems, my + o).wait_send()
        for j, o in enumerate(OFFS_B):
            @pl.when(my + o < N_DEV)
            def _(j=j, o=o):
                copy(sb_buf, rb_buf, j, sb_sems, rb_sems, my + o).wait_send()

        out_ref[:, :] = acc * e_ref[:, :]

    k = len(OFFS_A)
    return pl.pallas_call(
        body,
        out_shape=jax.ShapeDtypeStruct((m, n), jnp.float32),
        in_specs=[pl.BlockSpec(memory_space=pltpu.VMEM)],
        out_specs=pl.BlockSpec(memory_space=pltpu.VMEM),
        scratch_shapes=[
            pltpu.VMEM((1, n), jnp.float32),
            pltpu.VMEM((1, n), jnp.float32),
            pltpu.VMEM((1, n), jnp.float32),
            pltpu.VMEM((k, 1, n), jnp.float32),
            pltpu.VMEM((k, 1, n), jnp.float32),
            pltpu.SemaphoreType.DMA((k,)),
            pltpu.SemaphoreType.DMA((k,)),
            pltpu.SemaphoreType.DMA((k,)),
            pltpu.SemaphoreType.DMA((k,)),
            pltpu.SemaphoreType.REGULAR((k,)),
            pltpu.SemaphoreType.REGULAR((k,)),
        ],
        compiler_params=pltpu.CompilerParams(collective_id=0),
    )(x)
